# baseline (speedup 1.0000x reference)
import jax
import jax.numpy as jnp
from jax import lax
from jax.experimental import pallas as pl
from jax.experimental.pallas import tpu as pltpu
from jax.experimental.pallas import tpu_sc as plsc

M = 500000
C = 65536
B = 250000
D = 128

NC, NS = 2, 16
NW = NC * NS
Bp = 253952
PW = Bp // NW
GCH = 64
NCH = PW // GCH
SCH = Bp // NCH

KPW = 15632
R0F = 124928
FPW = R0F // NW
CPB = 2000
CPO = 62
RCH = 256
WCAP = 15872

_mesh = plsc.VectorSubcoreMesh(core_axis_name="c", subcore_axis_name="s")
_sc_params = pltpu.CompilerParams(needs_layout_passes=False)


def _wid():
    return lax.axis_index("s") * NC + lax.axis_index("c")


def _merged_body(prev_hbm, cfg_hbm, key_hbm, val_hbm,
                 psel_hbm, upd_hbm, wids_hbm, wkeys_hbm, nw_hbm,
                 kbuf, vbuf, keybufA, keybufB, lastidx,
                 rowsP0, rowsP1, rowsC0, rowsC1, wids, wkeys,
                 semK, semGP, semGC, semSP, semSC):
    w = _wid()
    ebase = w * PW
    kbase = w * KPW
    lane = lax.iota(jnp.int32, 16)
    keybuf2 = [keybufA, keybufB]
    rowsP = [rowsP0, rowsP1]
    rowsC = [rowsC0, rowsC1]

    def init(j, _):
        lastidx[pl.ds(j * 16, 16)] = jnp.full((16,), -1, jnp.int32)
        return 0

    lax.fori_loop(0, KPW // 16, init, 0)

    pltpu.sync_copy(key_hbm.at[pl.ds(ebase, PW)], kbuf)
    pltpu.sync_copy(val_hbm.at[pl.ds(ebase, PW)], vbuf)
    for b in range(2):
        pltpu.async_copy(key_hbm.at[pl.ds(b * SCH, SCH)], keybuf2[b],
                         semK.at[b])

    def _wait_gathers(c, bb):
        pltpu.make_async_copy(
            prev_hbm.at[kbuf.at[pl.ds(c * GCH, GCH)]], rowsP[bb],
            semGP.at[bb]).wait()
        pltpu.make_async_copy(
            cfg_hbm.at[vbuf.at[pl.ds(c * GCH, GCH)]], rowsC[bb],
            semGC.at[bb]).wait()

    def _issue_stores(c, bb):
        pltpu.async_copy(rowsP[bb],
                         psel_hbm.at[pl.ds(ebase + c * GCH, GCH)],
                         semSP.at[bb])
        pltpu.async_copy(rowsC[bb],
                         upd_hbm.at[pl.ds(ebase + c * GCH, GCH)],
                         semSC.at[bb])

    def _wait_stores(c, bb):
        pltpu.make_async_copy(rowsP[bb],
                              psel_hbm.at[pl.ds(ebase + c * GCH, GCH)],
                              semSP.at[bb]).wait()
        pltpu.make_async_copy(rowsC[bb],
                              upd_hbm.at[pl.ds(ebase + c * GCH, GCH)],
                              semSC.at[bb]).wait()

    def outer(o, _):
        for b in range(2):
            c = 2 * o + b
            pltpu.make_async_copy(key_hbm.at[pl.ds(c * SCH, SCH)],
                                  keybuf2[b], semK.at[b]).wait()

            @pl.when(o >= 1)
            def _():
                _wait_stores(c - 2, b)

            pltpu.async_copy(
                prev_hbm.at[kbuf.at[pl.ds(c * GCH, GCH)]], rowsP[b],
                semGP.at[b])
            pltpu.async_copy(
                cfg_hbm.at[vbuf.at[pl.ds(c * GCH, GCH)]], rowsC[b],
                semGC.at[b])

            kb = keybuf2[b]

            def vstep(j2, _):
                for u in range(4):
                    j = j2 * 4 + u
                    k16 = kb[pl.ds(j * 16, 16)]
                    ids = (c * SCH + j * 16) + lane
                    loc = k16 - kbase
                    inr = (loc >= 0) & (loc < KPW) & (ids < B)
                    plsc.store_scatter(lastidx, [loc], ids, mask=inr)
                return 0

            lax.fori_loop(0, SCH // 16 // 4, vstep, 0)

            @pl.when(c + 2 < NCH)
            def _():
                pltpu.async_copy(key_hbm.at[pl.ds((c + 2) * SCH, SCH)],
                                 keybuf2[b], semK.at[b])

            if b == 0:
                @pl.when(o >= 1)
                def _():
                    _wait_gathers(c - 1, 1)
                    _issue_stores(c - 1, 1)
            else:
                _wait_gathers(c - 1, 0)
                _issue_stores(c - 1, 0)
        return 0

    lax.fori_loop(0, NCH // 2, outer, 0)
    _wait_gathers(NCH - 1, 1)
    _issue_stores(NCH - 1, 1)
    _wait_stores(NCH - 2, 0)
    _wait_stores(NCH - 1, 1)

    def compress(j, off):
        li = lastidx[pl.ds(j * 16, 16)]
        m = li >= 0
        dk = (kbase + j * 16) + lane
        plsc.store_compressed(wids.at[pl.ds(off, 16)], li, mask=m)
        plsc.store_compressed(wkeys.at[pl.ds(off, 16)], dk, mask=m)
        return off + jnp.max(plsc.all_reduce_population_count(m))

    nw = lax.fori_loop(0, KPW // 16, compress, jnp.int32(0))

    @pl.when(nw > 0)
    def _():
        pad_sel = jnp.full((16,), nw - 1, jnp.int32)
        pid = plsc.load_gather(wids, [pad_sel])
        pkey = plsc.load_gather(wkeys, [pad_sel])
        p0 = (nw // 16) * 16
        mpad = (p0 + lane) >= nw
        plsc.store_scatter(wids, [p0 + lane], pid, mask=mpad)
        plsc.store_scatter(wkeys, [p0 + lane], pkey, mask=mpad)
        nwr = ((nw + 2 * RCH - 1) // (2 * RCH)) * (2 * RCH)

        def padv(q, _):
            wids[pl.ds(q * 16, 16)] = pid
            wkeys[pl.ds(q * 16, 16)] = pkey
            return 0

        lax.fori_loop(p0 // 16 + 1, nwr // 16, padv, 0)

    pltpu.sync_copy(wids, wids_hbm.at[pl.ds(w * WCAP, WCAP)])
    pltpu.sync_copy(wkeys, wkeys_hbm.at[pl.ds(w * WCAP, WCAP)])
    kb0 = keybufA
    kb0[pl.ds(0, 16)] = jnp.full((16,), nw, jnp.int32)
    pltpu.sync_copy(kb0.at[pl.ds(0, 16)], nw_hbm.at[pl.ds(w * 16, 16)])


_sc_merged = pl.kernel(
    _merged_body,
    out_type=(jax.ShapeDtypeStruct((Bp, D), jnp.float32),
              jax.ShapeDtypeStruct((Bp, D), jnp.float32),
              jax.ShapeDtypeStruct((NW * WCAP,), jnp.int32),
              jax.ShapeDtypeStruct((NW * WCAP,), jnp.int32),
              jax.ShapeDtypeStruct((NW * 16,), jnp.int32)),
    mesh=_mesh,
    scratch_types=[
        pltpu.VMEM((PW,), jnp.int32),
        pltpu.VMEM((PW,), jnp.int32),
        pltpu.VMEM((SCH,), jnp.int32),
        pltpu.VMEM((SCH,), jnp.int32),
        pltpu.VMEM((KPW,), jnp.int32),
        pltpu.VMEM((GCH, D), jnp.float32),
        pltpu.VMEM((GCH, D), jnp.float32),
        pltpu.VMEM((GCH, D), jnp.float32),
        pltpu.VMEM((GCH, D), jnp.float32),
        pltpu.VMEM((WCAP,), jnp.int32),
        pltpu.VMEM((WCAP,), jnp.int32),
        pltpu.SemaphoreType.DMA((2,)),
        pltpu.SemaphoreType.DMA((2,)),
        pltpu.SemaphoreType.DMA((2,)),
        pltpu.SemaphoreType.DMA((2,)),
        pltpu.SemaphoreType.DMA((2,)),
    ],
    compiler_params=_sc_params,
)


def _fill_body(prev_hbm, out_hbm, sem):
    w = _wid()
    base = w * FPW
    pltpu.async_copy(prev_hbm.at[pl.ds(base, FPW)],
                     out_hbm.at[pl.ds(base, FPW)], sem).wait()


_sc_fill = pl.kernel(
    _fill_body,
    out_type=(),
    mesh=_mesh,
    scratch_types=[pltpu.SemaphoreType.DMA],
    compiler_params=_sc_params,
)


def _scatter_body(wids_hbm, wkeys_hbm, nw_hbm, updates_hbm, out_hbm,
                  widv, wkeyv, nwbuf, rowbuf0, rowbuf1, semG, semS):
    rowbuf = [rowbuf0, rowbuf1]
    w = _wid()
    pltpu.sync_copy(wids_hbm.at[pl.ds(w * WCAP, WCAP)], widv)
    pltpu.sync_copy(wkeys_hbm.at[pl.ds(w * WCAP, WCAP)], wkeyv)
    pltpu.sync_copy(nw_hbm.at[pl.ds(w * 16, 16)], nwbuf)
    nw = jnp.max(nwbuf[...])

    @pl.when(nw > 0)
    def _():
        npair = (nw + 2 * RCH - 1) // (2 * RCH)

        def move(o, _):
            for b in range(2):
                c = 2 * o + b

                @pl.when(o >= 1)
                def _():
                    pltpu.make_async_copy(
                        rowbuf[b],
                        out_hbm.at[wkeyv.at[pl.ds((c - 2) * RCH, RCH)]],
                        semS.at[b]).wait()

                pltpu.async_copy(
                    updates_hbm.at[widv.at[pl.ds(c * RCH, RCH)]],
                    rowbuf[b], semG.at[b])
            for b in range(2):
                c = 2 * o + b
                pltpu.make_async_copy(
                    updates_hbm.at[widv.at[pl.ds(c * RCH, RCH)]],
                    rowbuf[b], semG.at[b]).wait()
                pltpu.async_copy(
                    rowbuf[b],
                    out_hbm.at[wkeyv.at[pl.ds(c * RCH, RCH)]],
                    semS.at[b])
            return 0

        lax.fori_loop(0, npair, move, 0)
        for b in range(2):
            c = 2 * (npair - 1) + b
            pltpu.make_async_copy(
                rowbuf[b],
                out_hbm.at[wkeyv.at[pl.ds(c * RCH, RCH)]],
                semS.at[b]).wait()


_sc_scatter = pl.kernel(
    _scatter_body,
    out_type=(),
    mesh=_mesh,
    scratch_types=[
        pltpu.VMEM((WCAP,), jnp.int32),
        pltpu.VMEM((WCAP,), jnp.int32),
        pltpu.VMEM((16,), jnp.int32),
        pltpu.VMEM((RCH, D), jnp.float32),
        pltpu.VMEM((RCH, D), jnp.float32),
        pltpu.SemaphoreType.DMA((2,)),
        pltpu.SemaphoreType.DMA((2,)),
    ],
    compiler_params=_sc_params,
)


def _copy_body(x_ref, o_ref):
    o_ref[...] = x_ref[...]


def _tc_copy_tail(x):
    return pl.pallas_call(
        _copy_body,
        grid=(M // CPB - CPO,),
        in_specs=[pl.BlockSpec((CPB, D), lambda i: (i + CPO, 0))],
        out_specs=pl.BlockSpec((CPB, D), lambda i: (i + CPO, 0)),
        out_shape=jax.ShapeDtypeStruct((M, D), jnp.float32),
    )(x)


UR = 4096


def _update_body(ps_ref, up_ref, wg_ref, bg_ref, wc_ref, bc_ref, o_ref):
    ps = ps_ref[...]
    up = up_ref[...]
    g = ps @ wg_ref[0:D, :] + up @ wg_ref[D:2 * D, :] + bg_ref[...]
    gate = jax.nn.sigmoid(g)
    c = ps @ wc_ref[0:D, :] + up @ wc_ref[D:2 * D, :] + bc_ref[...]
    cand = jnp.maximum(c, 0.0)
    o_ref[...] = gate * ps + (1.0 - gate) * cand


def _tc_update(ps, up, wg, bg, wc, bc):
    return pl.pallas_call(
        _update_body,
        grid=(Bp // UR,),
        in_specs=[
            pl.BlockSpec((UR, D), lambda i: (i, 0)),
            pl.BlockSpec((UR, D), lambda i: (i, 0)),
            pl.BlockSpec((2 * D, D), lambda i: (0, 0)),
            pl.BlockSpec((1, D), lambda i: (0, 0)),
            pl.BlockSpec((2 * D, D), lambda i: (0, 0)),
            pl.BlockSpec((1, D), lambda i: (0, 0)),
        ],
        out_specs=pl.BlockSpec((UR, D), lambda i: (i, 0)),
        out_shape=jax.ShapeDtypeStruct((Bp, D), jnp.float32),
    )(ps, up, wg, bg.reshape(1, D), wc, bc.reshape(1, D))


def kernel(previous_ast_nodes_encodings, new_cfg_nodes_encodings,
           key_indices, value_indices, W_gate, b_gate, W_cand, b_cand):
    pad = Bp - B
    padk = jnp.arange(pad, dtype=jnp.int32) % M
    padv = jnp.arange(pad, dtype=jnp.int32) % C
    kp = jnp.concatenate([key_indices.astype(jnp.int32), padk])
    vp = jnp.concatenate([value_indices.astype(jnp.int32), padv])

    psel, upd, wids, wkeys, nwv = _sc_merged(
        previous_ast_nodes_encodings, new_cfg_nodes_encodings, kp, vp)
    updates = _tc_update(psel, upd, W_gate, b_gate, W_cand, b_cand)

    out0 = _tc_copy_tail(previous_ast_nodes_encodings)
    oref = jax.new_ref(out0)
    _sc_fill(previous_ast_nodes_encodings, oref)
    _sc_scatter(wids, wkeys, nwv, updates, oref)
    return oref[...]


# speedup vs baseline: 3.6102x; 3.6102x over previous
import jax
import jax.numpy as jnp
from jax import lax
from jax.experimental import pallas as pl
from jax.experimental.pallas import tpu as pltpu
from jax.experimental.pallas import tpu_sc as plsc

M = 500000
C = 65536
B = 250000
D = 128

NC, NS = 2, 16
NW = NC * NS
Bp = 253952
PW = Bp // NW
GCH = 64
NCH = PW // GCH
SCH = Bp // NCH

KPW = 15632
R0F = 154880
FPW = R0F // NW
FCH = 440
CPB = 2000
CPO = 77
RCH = 256
WCAP = 15872

_mesh = plsc.VectorSubcoreMesh(core_axis_name="c", subcore_axis_name="s")
_sc_params = pltpu.CompilerParams(needs_layout_passes=False)


def _wid():
    return lax.axis_index("s") * NC + lax.axis_index("c")


def _merged_body(prev_hbm, cfg_hbm, key_hbm, val_hbm,
                 psel_hbm, upd_hbm, wids_hbm, wkeys_hbm, nw_hbm,
                 kbuf, vbuf, keybufA, keybufB, lastidx,
                 rowsP0, rowsP1, rowsC0, rowsC1, wids, wkeys,
                 semK, semGP, semGC, semSP, semSC):
    w = _wid()
    ebase = w * PW
    kbase = w * KPW
    lane = lax.iota(jnp.int32, 16)
    keybuf2 = [keybufA, keybufB]
    rowsP = [rowsP0, rowsP1]
    rowsC = [rowsC0, rowsC1]

    def init(j, _):
        lastidx[pl.ds(j * 16, 16)] = jnp.full((16,), -1, jnp.int32)
        return 0

    lax.fori_loop(0, KPW // 16, init, 0)

    pltpu.sync_copy(key_hbm.at[pl.ds(ebase, PW)], kbuf)
    pltpu.sync_copy(val_hbm.at[pl.ds(ebase, PW)], vbuf)
    for b in range(2):
        pltpu.async_copy(key_hbm.at[pl.ds(b * SCH, SCH)], keybuf2[b],
                         semK.at[b])

    def _wait_gathers(c, bb):
        pltpu.make_async_copy(
            prev_hbm.at[kbuf.at[pl.ds(c * GCH, GCH)]], rowsP[bb],
            semGP.at[bb]).wait()
        pltpu.make_async_copy(
            cfg_hbm.at[vbuf.at[pl.ds(c * GCH, GCH)]], rowsC[bb],
            semGC.at[bb]).wait()

    def _issue_stores(c, bb):
        pltpu.async_copy(rowsP[bb],
                         psel_hbm.at[pl.ds(ebase + c * GCH, GCH)],
                         semSP.at[bb])
        pltpu.async_copy(rowsC[bb],
                         upd_hbm.at[pl.ds(ebase + c * GCH, GCH)],
                         semSC.at[bb])

    def _wait_stores(c, bb):
        pltpu.make_async_copy(rowsP[bb],
                              psel_hbm.at[pl.ds(ebase + c * GCH, GCH)],
                              semSP.at[bb]).wait()
        pltpu.make_async_copy(rowsC[bb],
                              upd_hbm.at[pl.ds(ebase + c * GCH, GCH)],
                              semSC.at[bb]).wait()

    def outer(o, _):
        for b in range(2):
            c = 2 * o + b
            pltpu.make_async_copy(key_hbm.at[pl.ds(c * SCH, SCH)],
                                  keybuf2[b], semK.at[b]).wait()

            @pl.when(o >= 1)
            def _():
                _wait_stores(c - 2, b)

            pltpu.async_copy(
                prev_hbm.at[kbuf.at[pl.ds(c * GCH, GCH)]], rowsP[b],
                semGP.at[b])
            pltpu.async_copy(
                cfg_hbm.at[vbuf.at[pl.ds(c * GCH, GCH)]], rowsC[b],
                semGC.at[b])

            kb = keybuf2[b]

            def vstep(j2, _):
                for u in range(4):
                    j = j2 * 4 + u
                    k16 = kb[pl.ds(j * 16, 16)]
                    ids = (c * SCH + j * 16) + lane
                    loc = k16 - kbase
                    inr = (loc >= 0) & (loc < KPW) & (ids < B)
                    plsc.store_scatter(lastidx, [loc], ids, mask=inr)
                return 0

            lax.fori_loop(0, SCH // 16 // 4, vstep, 0)

            @pl.when(c + 2 < NCH)
            def _():
                pltpu.async_copy(key_hbm.at[pl.ds((c + 2) * SCH, SCH)],
                                 keybuf2[b], semK.at[b])

            if b == 0:
                @pl.when(o >= 1)
                def _():
                    _wait_gathers(c - 1, 1)
                    _issue_stores(c - 1, 1)
            else:
                _wait_gathers(c - 1, 0)
                _issue_stores(c - 1, 0)
        return 0

    lax.fori_loop(0, NCH // 2, outer, 0)
    _wait_gathers(NCH - 1, 1)
    _issue_stores(NCH - 1, 1)
    _wait_stores(NCH - 2, 0)
    _wait_stores(NCH - 1, 1)

    def compress(j, off):
        li = lastidx[pl.ds(j * 16, 16)]
        m = li >= 0
        dk = (kbase + j * 16) + lane
        plsc.store_compressed(wids.at[pl.ds(off, 16)], li, mask=m)
        plsc.store_compressed(wkeys.at[pl.ds(off, 16)], dk, mask=m)
        return off + jnp.max(plsc.all_reduce_population_count(m))

    nw = lax.fori_loop(0, KPW // 16, compress, jnp.int32(0))

    @pl.when(nw > 0)
    def _():
        pad_sel = jnp.full((16,), nw - 1, jnp.int32)
        pid = plsc.load_gather(wids, [pad_sel])
        pkey = plsc.load_gather(wkeys, [pad_sel])
        p0 = (nw // 16) * 16
        mpad = (p0 + lane) >= nw
        plsc.store_scatter(wids, [p0 + lane], pid, mask=mpad)
        plsc.store_scatter(wkeys, [p0 + lane], pkey, mask=mpad)
        nwr = ((nw + 2 * RCH - 1) // (2 * RCH)) * (2 * RCH)

        def padv(q, _):
            wids[pl.ds(q * 16, 16)] = pid
            wkeys[pl.ds(q * 16, 16)] = pkey
            return 0

        lax.fori_loop(p0 // 16 + 1, nwr // 16, padv, 0)

    pltpu.sync_copy(wids, wids_hbm.at[pl.ds(w * WCAP, WCAP)])
    pltpu.sync_copy(wkeys, wkeys_hbm.at[pl.ds(w * WCAP, WCAP)])
    kb0 = keybufA
    kb0[pl.ds(0, 16)] = jnp.full((16,), nw, jnp.int32)
    pltpu.sync_copy(kb0.at[pl.ds(0, 16)], nw_hbm.at[pl.ds(w * 16, 16)])


_sc_merged = pl.kernel(
    _merged_body,
    out_type=(jax.ShapeDtypeStruct((Bp, D), jnp.float32),
              jax.ShapeDtypeStruct((Bp, D), jnp.float32),
              jax.ShapeDtypeStruct((NW * WCAP,), jnp.int32),
              jax.ShapeDtypeStruct((NW * WCAP,), jnp.int32),
              jax.ShapeDtypeStruct((NW * 16,), jnp.int32)),
    mesh=_mesh,
    scratch_types=[
        pltpu.VMEM((PW,), jnp.int32),
        pltpu.VMEM((PW,), jnp.int32),
        pltpu.VMEM((SCH,), jnp.int32),
        pltpu.VMEM((SCH,), jnp.int32),
        pltpu.VMEM((KPW,), jnp.int32),
        pltpu.VMEM((GCH, D), jnp.float32),
        pltpu.VMEM((GCH, D), jnp.float32),
        pltpu.VMEM((GCH, D), jnp.float32),
        pltpu.VMEM((GCH, D), jnp.float32),
        pltpu.VMEM((WCAP,), jnp.int32),
        pltpu.VMEM((WCAP,), jnp.int32),
        pltpu.SemaphoreType.DMA((2,)),
        pltpu.SemaphoreType.DMA((2,)),
        pltpu.SemaphoreType.DMA((2,)),
        pltpu.SemaphoreType.DMA((2,)),
        pltpu.SemaphoreType.DMA((2,)),
    ],
    compiler_params=_sc_params,
)


def _fill_body(prev_hbm, out_hbm, fb0, fb1, semR, semW):
    w = _wid()
    base = w * FPW
    fb = [fb0, fb1]
    nch = FPW // FCH
    for c in range(nch):
        b = c & 1
        lo = base + c * FCH
        if c >= 2:
            pltpu.make_async_copy(fb[b], out_hbm.at[pl.ds(lo - 2 * FCH, FCH)],
                                  semW.at[b]).wait()
        pltpu.async_copy(prev_hbm.at[pl.ds(lo, FCH)], fb[b],
                         semR.at[b]).wait()
        pltpu.async_copy(fb[b], out_hbm.at[pl.ds(lo, FCH)], semW.at[b])
    for c in range(nch - 2, nch):
        b = c & 1
        lo = base + c * FCH
        pltpu.make_async_copy(fb[b], out_hbm.at[pl.ds(lo, FCH)],
                              semW.at[b]).wait()


_sc_fill = pl.kernel(
    _fill_body,
    out_type=(),
    mesh=_mesh,
    scratch_types=[
        pltpu.VMEM((FCH, D), jnp.float32),
        pltpu.VMEM((FCH, D), jnp.float32),
        pltpu.SemaphoreType.DMA((2,)),
        pltpu.SemaphoreType.DMA((2,)),
    ],
    compiler_params=_sc_params,
)


def _scatter_body(wids_hbm, wkeys_hbm, nw_hbm, updates_hbm, out_hbm,
                  widv, wkeyv, nwbuf, rowbuf0, rowbuf1, semG, semS):
    rowbuf = [rowbuf0, rowbuf1]
    w = _wid()
    pltpu.sync_copy(wids_hbm.at[pl.ds(w * WCAP, WCAP)], widv)
    pltpu.sync_copy(wkeys_hbm.at[pl.ds(w * WCAP, WCAP)], wkeyv)
    pltpu.sync_copy(nw_hbm.at[pl.ds(w * 16, 16)], nwbuf)
    nw = jnp.max(nwbuf[...])

    @pl.when(nw > 0)
    def _():
        npair = (nw + 2 * RCH - 1) // (2 * RCH)

        def move(o, _):
            for b in range(2):
                c = 2 * o + b

                @pl.when(o >= 1)
                def _():
                    pltpu.make_async_copy(
                        rowbuf[b],
                        out_hbm.at[wkeyv.at[pl.ds((c - 2) * RCH, RCH)]],
                        semS.at[b]).wait()

                pltpu.async_copy(
                    updates_hbm.at[widv.at[pl.ds(c * RCH, RCH)]],
                    rowbuf[b], semG.at[b])
            for b in range(2):
                c = 2 * o + b
                pltpu.make_async_copy(
                    updates_hbm.at[widv.at[pl.ds(c * RCH, RCH)]],
                    rowbuf[b], semG.at[b]).wait()
                pltpu.async_copy(
                    rowbuf[b],
                    out_hbm.at[wkeyv.at[pl.ds(c * RCH, RCH)]],
                    semS.at[b])
            return 0

        lax.fori_loop(0, npair, move, 0)
        for b in range(2):
            c = 2 * (npair - 1) + b
            pltpu.make_async_copy(
                rowbuf[b],
                out_hbm.at[wkeyv.at[pl.ds(c * RCH, RCH)]],
                semS.at[b]).wait()


_sc_scatter = pl.kernel(
    _scatter_body,
    out_type=(),
    mesh=_mesh,
    scratch_types=[
        pltpu.VMEM((WCAP,), jnp.int32),
        pltpu.VMEM((WCAP,), jnp.int32),
        pltpu.VMEM((16,), jnp.int32),
        pltpu.VMEM((RCH, D), jnp.float32),
        pltpu.VMEM((RCH, D), jnp.float32),
        pltpu.SemaphoreType.DMA((2,)),
        pltpu.SemaphoreType.DMA((2,)),
    ],
    compiler_params=_sc_params,
)


def _copy_body(x_ref, o_ref):
    o_ref[...] = x_ref[...]


def _tc_copy_tail(x):
    return pl.pallas_call(
        _copy_body,
        grid=(M // CPB - CPO,),
        in_specs=[pl.BlockSpec((CPB, D), lambda i: (i + CPO, 0))],
        out_specs=pl.BlockSpec((CPB, D), lambda i: (i + CPO, 0)),
        out_shape=jax.ShapeDtypeStruct((M, D), jnp.float32),
    )(x)


UR = 4096


def _update_body(ps_ref, up_ref, wg_ref, bg_ref, wc_ref, bc_ref, o_ref):
    ps = ps_ref[...]
    up = up_ref[...]
    g = ps @ wg_ref[0:D, :] + up @ wg_ref[D:2 * D, :] + bg_ref[...]
    gate = jax.nn.sigmoid(g)
    c = ps @ wc_ref[0:D, :] + up @ wc_ref[D:2 * D, :] + bc_ref[...]
    cand = jnp.maximum(c, 0.0)
    o_ref[...] = gate * ps + (1.0 - gate) * cand


def _tc_update(ps, up, wg, bg, wc, bc):
    return pl.pallas_call(
        _update_body,
        grid=(Bp // UR,),
        in_specs=[
            pl.BlockSpec((UR, D), lambda i: (i, 0)),
            pl.BlockSpec((UR, D), lambda i: (i, 0)),
            pl.BlockSpec((2 * D, D), lambda i: (0, 0)),
            pl.BlockSpec((1, D), lambda i: (0, 0)),
            pl.BlockSpec((2 * D, D), lambda i: (0, 0)),
            pl.BlockSpec((1, D), lambda i: (0, 0)),
        ],
        out_specs=pl.BlockSpec((UR, D), lambda i: (i, 0)),
        out_shape=jax.ShapeDtypeStruct((Bp, D), jnp.float32),
    )(ps, up, wg, bg.reshape(1, D), wc, bc.reshape(1, D))


def kernel(previous_ast_nodes_encodings, new_cfg_nodes_encodings,
           key_indices, value_indices, W_gate, b_gate, W_cand, b_cand):
    pad = Bp - B
    padk = jnp.arange(pad, dtype=jnp.int32) % M
    padv = jnp.arange(pad, dtype=jnp.int32) % C
    kp = jnp.concatenate([key_indices.astype(jnp.int32), padk])
    vp = jnp.concatenate([value_indices.astype(jnp.int32), padv])

    psel, upd, wids, wkeys, nwv = _sc_merged(
        previous_ast_nodes_encodings, new_cfg_nodes_encodings, kp, vp)
    updates = _tc_update(psel, upd, W_gate, b_gate, W_cand, b_cand)

    out0 = _tc_copy_tail(previous_ast_nodes_encodings)
    oref = jax.new_ref(out0)
    _sc_fill(previous_ast_nodes_encodings, oref)
    _sc_scatter(wids, wkeys, nwv, updates, oref)
    return oref[...]


# speedup vs baseline: 3.6874x; 1.0214x over previous
import jax
import jax.numpy as jnp
from jax import lax
from jax.experimental import pallas as pl
from jax.experimental.pallas import tpu as pltpu
from jax.experimental.pallas import tpu_sc as plsc

M = 500000
C = 65536
B = 250000
D = 128

NC, NS = 2, 16
NW = NC * NS
Bp = 253952
PW = Bp // NW
GCH = 64
NCH = PW // GCH
SCH = Bp // NCH

KPW = 15632
R0F = 154880
FPW = R0F // NW
FCH = 440
CPB = 2000
CPO = 77
RCH = 128
NBUF = 4
WCAP = 15872

_mesh = plsc.VectorSubcoreMesh(core_axis_name="c", subcore_axis_name="s")
_sc_params = pltpu.CompilerParams(needs_layout_passes=False)


def _wid():
    return lax.axis_index("s") * NC + lax.axis_index("c")


def _merged_body(prev_hbm, cfg_hbm, key_hbm, val_hbm,
                 psel_hbm, upd_hbm, wids_hbm, wkeys_hbm, nw_hbm,
                 kbuf, vbuf, keybufA, keybufB, lastidx,
                 rowsP0, rowsP1, rowsC0, rowsC1, wids, wkeys,
                 semK, semGP, semGC, semSP, semSC):
    w = _wid()
    ebase = w * PW
    kbase = w * KPW
    lane = lax.iota(jnp.int32, 16)
    keybuf2 = [keybufA, keybufB]
    rowsP = [rowsP0, rowsP1]
    rowsC = [rowsC0, rowsC1]

    def init(j, _):
        lastidx[pl.ds(j * 16, 16)] = jnp.full((16,), -1, jnp.int32)
        return 0

    lax.fori_loop(0, KPW // 16, init, 0)

    pltpu.sync_copy(key_hbm.at[pl.ds(ebase, PW)], kbuf)
    pltpu.sync_copy(val_hbm.at[pl.ds(ebase, PW)], vbuf)
    for b in range(2):
        pltpu.async_copy(key_hbm.at[pl.ds(b * SCH, SCH)], keybuf2[b],
                         semK.at[b])

    def _wait_gathers(c, bb):
        pltpu.make_async_copy(
            prev_hbm.at[kbuf.at[pl.ds(c * GCH, GCH)]], rowsP[bb],
            semGP.at[bb]).wait()
        pltpu.make_async_copy(
            cfg_hbm.at[vbuf.at[pl.ds(c * GCH, GCH)]], rowsC[bb],
            semGC.at[bb]).wait()

    def _issue_stores(c, bb):
        pltpu.async_copy(rowsP[bb],
                         psel_hbm.at[pl.ds(ebase + c * GCH, GCH)],
                         semSP.at[bb])
        pltpu.async_copy(rowsC[bb],
                         upd_hbm.at[pl.ds(ebase + c * GCH, GCH)],
                         semSC.at[bb])

    def _wait_stores(c, bb):
        pltpu.make_async_copy(rowsP[bb],
                              psel_hbm.at[pl.ds(ebase + c * GCH, GCH)],
                              semSP.at[bb]).wait()
        pltpu.make_async_copy(rowsC[bb],
                              upd_hbm.at[pl.ds(ebase + c * GCH, GCH)],
                              semSC.at[bb]).wait()

    def outer(o, _):
        for b in range(2):
            c = 2 * o + b
            pltpu.make_async_copy(key_hbm.at[pl.ds(c * SCH, SCH)],
                                  keybuf2[b], semK.at[b]).wait()

            @pl.when(o >= 1)
            def _():
                _wait_stores(c - 2, b)

            pltpu.async_copy(
                prev_hbm.at[kbuf.at[pl.ds(c * GCH, GCH)]], rowsP[b],
                semGP.at[b])
            pltpu.async_copy(
                cfg_hbm.at[vbuf.at[pl.ds(c * GCH, GCH)]], rowsC[b],
                semGC.at[b])

            kb = keybuf2[b]

            def vstep(j2, _):
                for u in range(4):
                    j = j2 * 4 + u
                    k16 = kb[pl.ds(j * 16, 16)]
                    ids = (c * SCH + j * 16) + lane
                    loc = k16 - kbase
                    inr = (loc >= 0) & (loc < KPW) & (ids < B)
                    plsc.store_scatter(lastidx, [loc], ids, mask=inr)
                return 0

            lax.fori_loop(0, SCH // 16 // 4, vstep, 0)

            @pl.when(c + 2 < NCH)
            def _():
                pltpu.async_copy(key_hbm.at[pl.ds((c + 2) * SCH, SCH)],
                                 keybuf2[b], semK.at[b])

            if b == 0:
                @pl.when(o >= 1)
                def _():
                    _wait_gathers(c - 1, 1)
                    _issue_stores(c - 1, 1)
            else:
                _wait_gathers(c - 1, 0)
                _issue_stores(c - 1, 0)
        return 0

    lax.fori_loop(0, NCH // 2, outer, 0)
    _wait_gathers(NCH - 1, 1)
    _issue_stores(NCH - 1, 1)
    _wait_stores(NCH - 2, 0)
    _wait_stores(NCH - 1, 1)

    def compress(j, off):
        li = lastidx[pl.ds(j * 16, 16)]
        m = li >= 0
        dk = (kbase + j * 16) + lane
        plsc.store_compressed(wids.at[pl.ds(off, 16)], li, mask=m)
        plsc.store_compressed(wkeys.at[pl.ds(off, 16)], dk, mask=m)
        return off + jnp.max(plsc.all_reduce_population_count(m))

    nw = lax.fori_loop(0, KPW // 16, compress, jnp.int32(0))

    @pl.when(nw > 0)
    def _():
        pad_sel = jnp.full((16,), nw - 1, jnp.int32)
        pid = plsc.load_gather(wids, [pad_sel])
        pkey = plsc.load_gather(wkeys, [pad_sel])
        p0 = (nw // 16) * 16
        mpad = (p0 + lane) >= nw
        plsc.store_scatter(wids, [p0 + lane], pid, mask=mpad)
        plsc.store_scatter(wkeys, [p0 + lane], pkey, mask=mpad)
        nwr = ((nw + NBUF * RCH - 1) // (NBUF * RCH)) * (NBUF * RCH)

        def padv(q, _):
            wids[pl.ds(q * 16, 16)] = pid
            wkeys[pl.ds(q * 16, 16)] = pkey
            return 0

        lax.fori_loop(p0 // 16 + 1, nwr // 16, padv, 0)

    pltpu.sync_copy(wids, wids_hbm.at[pl.ds(w * WCAP, WCAP)])
    pltpu.sync_copy(wkeys, wkeys_hbm.at[pl.ds(w * WCAP, WCAP)])
    kb0 = keybufA
    kb0[pl.ds(0, 16)] = jnp.full((16,), nw, jnp.int32)
    pltpu.sync_copy(kb0.at[pl.ds(0, 16)], nw_hbm.at[pl.ds(w * 16, 16)])


_sc_merged = pl.kernel(
    _merged_body,
    out_type=(jax.ShapeDtypeStruct((Bp, D), jnp.float32),
              jax.ShapeDtypeStruct((Bp, D), jnp.float32),
              jax.ShapeDtypeStruct((NW * WCAP,), jnp.int32),
              jax.ShapeDtypeStruct((NW * WCAP,), jnp.int32),
              jax.ShapeDtypeStruct((NW * 16,), jnp.int32)),
    mesh=_mesh,
    scratch_types=[
        pltpu.VMEM((PW,), jnp.int32),
        pltpu.VMEM((PW,), jnp.int32),
        pltpu.VMEM((SCH,), jnp.int32),
        pltpu.VMEM((SCH,), jnp.int32),
        pltpu.VMEM((KPW,), jnp.int32),
        pltpu.VMEM((GCH, D), jnp.float32),
        pltpu.VMEM((GCH, D), jnp.float32),
        pltpu.VMEM((GCH, D), jnp.float32),
        pltpu.VMEM((GCH, D), jnp.float32),
        pltpu.VMEM((WCAP,), jnp.int32),
        pltpu.VMEM((WCAP,), jnp.int32),
        pltpu.SemaphoreType.DMA((2,)),
        pltpu.SemaphoreType.DMA((2,)),
        pltpu.SemaphoreType.DMA((2,)),
        pltpu.SemaphoreType.DMA((2,)),
        pltpu.SemaphoreType.DMA((2,)),
    ],
    compiler_params=_sc_params,
)


def _fill_body(prev_hbm, out_hbm, fb0, fb1, semR, semW):
    w = _wid()
    base = w * FPW
    fb = [fb0, fb1]
    nch = FPW // FCH
    for c in range(nch):
        b = c & 1
        lo = base + c * FCH
        if c >= 2:
            pltpu.make_async_copy(fb[b], out_hbm.at[pl.ds(lo - 2 * FCH, FCH)],
                                  semW.at[b]).wait()
        pltpu.async_copy(prev_hbm.at[pl.ds(lo, FCH)], fb[b],
                         semR.at[b]).wait()
        pltpu.async_copy(fb[b], out_hbm.at[pl.ds(lo, FCH)], semW.at[b])
    for c in range(nch - 2, nch):
        b = c & 1
        lo = base + c * FCH
        pltpu.make_async_copy(fb[b], out_hbm.at[pl.ds(lo, FCH)],
                              semW.at[b]).wait()


_sc_fill = pl.kernel(
    _fill_body,
    out_type=(),
    mesh=_mesh,
    scratch_types=[
        pltpu.VMEM((FCH, D), jnp.float32),
        pltpu.VMEM((FCH, D), jnp.float32),
        pltpu.SemaphoreType.DMA((2,)),
        pltpu.SemaphoreType.DMA((2,)),
    ],
    compiler_params=_sc_params,
)


def _scatter_body(wids_hbm, wkeys_hbm, nw_hbm, updates_hbm, out_hbm,
                  widv, wkeyv, nwbuf, rb0, rb1, rb2, rb3, semG, semS):
    rowbuf = [rb0, rb1, rb2, rb3]
    w = _wid()
    pltpu.sync_copy(wids_hbm.at[pl.ds(w * WCAP, WCAP)], widv)
    pltpu.sync_copy(wkeys_hbm.at[pl.ds(w * WCAP, WCAP)], wkeyv)
    pltpu.sync_copy(nw_hbm.at[pl.ds(w * 16, 16)], nwbuf)
    nw = jnp.max(nwbuf[...])

    @pl.when(nw > 0)
    def _():
        ngrp = (nw + NBUF * RCH - 1) // (NBUF * RCH)

        def move(o, _):
            for b in range(NBUF):
                c = NBUF * o + b

                @pl.when(o >= 1)
                def _():
                    pltpu.make_async_copy(
                        rowbuf[b],
                        out_hbm.at[wkeyv.at[pl.ds((c - NBUF) * RCH, RCH)]],
                        semS.at[b]).wait()

                pltpu.async_copy(
                    updates_hbm.at[widv.at[pl.ds(c * RCH, RCH)]],
                    rowbuf[b], semG.at[b])
            for b in range(NBUF):
                c = NBUF * o + b
                pltpu.make_async_copy(
                    updates_hbm.at[widv.at[pl.ds(c * RCH, RCH)]],
                    rowbuf[b], semG.at[b]).wait()
                pltpu.async_copy(
                    rowbuf[b],
                    out_hbm.at[wkeyv.at[pl.ds(c * RCH, RCH)]],
                    semS.at[b])
            return 0

        lax.fori_loop(0, ngrp, move, 0)
        for b in range(NBUF):
            c = NBUF * (ngrp - 1) + b
            pltpu.make_async_copy(
                rowbuf[b],
                out_hbm.at[wkeyv.at[pl.ds(c * RCH, RCH)]],
                semS.at[b]).wait()


_sc_scatter = pl.kernel(
    _scatter_body,
    out_type=(),
    mesh=_mesh,
    scratch_types=[
        pltpu.VMEM((WCAP,), jnp.int32),
        pltpu.VMEM((WCAP,), jnp.int32),
        pltpu.VMEM((16,), jnp.int32),
        pltpu.VMEM((RCH, D), jnp.float32),
        pltpu.VMEM((RCH, D), jnp.float32),
        pltpu.VMEM((RCH, D), jnp.float32),
        pltpu.VMEM((RCH, D), jnp.float32),
        pltpu.SemaphoreType.DMA((NBUF,)),
        pltpu.SemaphoreType.DMA((NBUF,)),
    ],
    compiler_params=_sc_params,
)


def _copy_body(x_ref, o_ref):
    o_ref[...] = x_ref[...]


def _tc_copy_tail(x):
    return pl.pallas_call(
        _copy_body,
        grid=(M // CPB - CPO,),
        in_specs=[pl.BlockSpec((CPB, D), lambda i: (i + CPO, 0))],
        out_specs=pl.BlockSpec((CPB, D), lambda i: (i + CPO, 0)),
        out_shape=jax.ShapeDtypeStruct((M, D), jnp.float32),
    )(x)


UR = 8192


def _update_body(ps_ref, up_ref, wg_ref, bg_ref, wc_ref, bc_ref, o_ref):
    ps = ps_ref[...]
    up = up_ref[...]
    g = ps @ wg_ref[0:D, :] + up @ wg_ref[D:2 * D, :] + bg_ref[...]
    gate = jax.nn.sigmoid(g)
    c = ps @ wc_ref[0:D, :] + up @ wc_ref[D:2 * D, :] + bc_ref[...]
    cand = jnp.maximum(c, 0.0)
    o_ref[...] = gate * ps + (1.0 - gate) * cand


def _tc_update(ps, up, wg, bg, wc, bc):
    return pl.pallas_call(
        _update_body,
        grid=(Bp // UR,),
        in_specs=[
            pl.BlockSpec((UR, D), lambda i: (i, 0)),
            pl.BlockSpec((UR, D), lambda i: (i, 0)),
            pl.BlockSpec((2 * D, D), lambda i: (0, 0)),
            pl.BlockSpec((1, D), lambda i: (0, 0)),
            pl.BlockSpec((2 * D, D), lambda i: (0, 0)),
            pl.BlockSpec((1, D), lambda i: (0, 0)),
        ],
        out_specs=pl.BlockSpec((UR, D), lambda i: (i, 0)),
        out_shape=jax.ShapeDtypeStruct((Bp, D), jnp.float32),
    )(ps, up, wg, bg.reshape(1, D), wc, bc.reshape(1, D))


def kernel(previous_ast_nodes_encodings, new_cfg_nodes_encodings,
           key_indices, value_indices, W_gate, b_gate, W_cand, b_cand):
    pad = Bp - B
    padk = jnp.arange(pad, dtype=jnp.int32) % M
    padv = jnp.arange(pad, dtype=jnp.int32) % C
    kp = jnp.concatenate([key_indices.astype(jnp.int32), padk])
    vp = jnp.concatenate([value_indices.astype(jnp.int32), padv])

    psel, upd, wids, wkeys, nwv = _sc_merged(
        previous_ast_nodes_encodings, new_cfg_nodes_encodings, kp, vp)
    updates = _tc_update(psel, upd, W_gate, b_gate, W_cand, b_cand)

    out0 = _tc_copy_tail(previous_ast_nodes_encodings)
    oref = jax.new_ref(out0)
    _sc_fill(previous_ast_nodes_encodings, oref)
    _sc_scatter(wids, wkeys, nwv, updates, oref)
    return oref[...]
